# baseline (speedup 1.0000x reference)
import functools

import jax
import jax.numpy as jnp
from jax import lax
from jax.experimental import pallas as pl
from jax.experimental.pallas import tpu as pltpu
from jax.experimental.pallas import tpu_sc as plsc

N = 10000
E = 160000
IN_DIM = 256
HID = 512
G = 128

NC, NS, LANES = 2, 16, 16
EB = 128
EP = 163840
EPT = EP // NS
NB_AGG = EPT // EB
EPW = EP // (NC * NS)
NB_DEG = EPW // EB
ACC_ROWS = 10112
RPT = ACC_ROWS // NS
RB = 1000
NRB = N // RB


def _sc_mesh():
    return plsc.VectorSubcoreMesh(
        core_axis_name="c", subcore_axis_name="s", num_cores=NC, num_subcores=NS
    )


def _make_deg_kernel():
    @functools.partial(
        pl.kernel,
        out_type=jax.ShapeDtypeStruct((NC * ACC_ROWS, 128), jnp.float32),
        mesh=_sc_mesh(),
        scratch_types=[
            pltpu.VMEM((NB_DEG, EB), jnp.int32),
            pltpu.VMEM((EB, 128), jnp.float32),
            pltpu.VMEM_SHARED((ACC_ROWS, 128), jnp.float32),
        ],
    )
    def deg_kernel(dst_hbm, ones_hbm, zeros_hbm, out_hbm, dst_v, ones_v, acc):
        c = lax.axis_index("c")
        s = lax.axis_index("s")
        w = c * NS + s
        pltpu.sync_copy(dst_hbm.at[w], dst_v)
        pltpu.sync_copy(ones_hbm, ones_v)
        pltpu.sync_copy(zeros_hbm, acc.at[pl.ds(s * RPT, RPT)])
        plsc.subcore_barrier()

        def body(b, carry):
            pltpu.sync_copy(ones_v, acc.at[dst_v.at[b]], add=True)
            return carry

        lax.fori_loop(0, NB_DEG, body, 0)
        plsc.subcore_barrier()
        pltpu.sync_copy(
            acc.at[pl.ds(s * RPT, RPT)],
            out_hbm.at[pl.ds(c * ACC_ROWS + s * RPT, RPT)],
        )

    return deg_kernel


EB_A = 64
NB_A = EPT // EB_A
NQ = 4
NB_Q = NB_A // NQ
NBUF = 2


def _pack_bf16_rows(x):
    r = x.shape[0]
    blk = x.reshape(r, 4, 2, 16)
    a = blk[:, :, 0, :].astype(jnp.bfloat16)
    b = blk[:, :, 1, :].astype(jnp.bfloat16)
    au = jax.lax.bitcast_convert_type(a, jnp.uint16).astype(jnp.uint32)
    bu = jax.lax.bitcast_convert_type(b, jnp.uint16).astype(jnp.uint32)
    w = au | (bu << 16)
    return jax.lax.bitcast_convert_type(w, jnp.int32).reshape(r, 64)


def _make_agg_kernel(n_chunks):
    cpc = n_chunks // NC

    @functools.partial(
        pl.kernel,
        out_type=jax.ShapeDtypeStruct((n_chunks * N, 128), jnp.float32),
        mesh=_sc_mesh(),
        compiler_params=pltpu.CompilerParams(use_tc_tiling_on_sc=False),
        scratch_types=[
            pltpu.VMEM((NB_Q, EB_A), jnp.int32),
            pltpu.VMEM((NB_Q, EB_A), jnp.int32),
            [pltpu.VMEM((EB_A, 64), jnp.int32) for _ in range(NBUF)],
            [pltpu.VMEM((EB_A, 128), jnp.float32) for _ in range(NBUF)],
            [pltpu.SemaphoreType.DMA for _ in range(NBUF)],
            [pltpu.SemaphoreType.DMA for _ in range(NBUF)],
            pltpu.VMEM_SHARED((ACC_ROWS, 128), jnp.float32),
        ],
    )
    def agg_kernel(table_hbm, src_hbm, dst_hbm, zeros_hbm, out_hbm,
                   src_v, dst_v, gbufs, fbufs, gsems, ssems, acc):
        c = lax.axis_index("c")
        s = lax.axis_index("s")

        def gath(b, k):
            return pltpu.make_async_copy(
                table_hbm.at[src_v.at[b]], gbufs[k], gsems[k]
            )

        class _Scat:
            def __init__(self, b, k):
                self.b, self.k = b, k

            def start(self):
                pltpu.async_copy(
                    fbufs[self.k], acc.at[dst_v.at[self.b]], ssems[self.k],
                    add=True,
                )

            def wait(self):
                pltpu.make_async_copy(
                    fbufs[self.k], acc.at[dst_v.at[self.b]], ssems[self.k]
                ).wait()

        scat = _Scat

        def conv(k):
            def cbody(r8, carry):
                for rr in range(8):
                    r = r8 * 8 + rr
                    for gq in range(4):
                        col = gq * 16
                        v32 = gbufs[k][r, pl.ds(col, 16)]
                        lo = jax.lax.bitcast_convert_type(
                            v32 << 16, jnp.float32)
                        hi = jax.lax.bitcast_convert_type(
                            v32 & jnp.int32(-65536), jnp.float32)
                        fbufs[k][r, pl.ds(2 * col, 16)] = lo
                        fbufs[k][r, pl.ds(2 * col + 16, 16)] = hi
                return carry

            lax.fori_loop(0, EB_A // 8, cbody, 0)

        for j in range(cpc):
            off = (c * cpc + j) * N

            pltpu.sync_copy(zeros_hbm, acc.at[pl.ds(s * RPT, RPT)])
            plsc.subcore_barrier()

            def qbody(q, qcarry):
                seg = s * NQ + q
                pltpu.sync_copy(src_hbm.at[seg], src_v)
                pltpu.sync_copy(dst_hbm.at[seg], dst_v)

                def obody(i, carry):
                    r = i // (EB_A // LANES)
                    col = (i % (EB_A // LANES)) * LANES
                    src_v[r, pl.ds(col, LANES)] = (
                        src_v[r, pl.ds(col, LANES)] + off
                    )
                    return carry

                lax.fori_loop(0, NB_Q * (EB_A // LANES), obody, 0)

                gath(0, 0).start()

                def body(bb, carry):
                    for par in range(NBUF):
                        b = bb * NBUF + par

                        @pl.when(b + 1 < NB_Q)
                        def _():
                            gath(b + 1, (par + 1) % NBUF).start()

                        gath(b, par).wait()

                        @pl.when(b >= NBUF)
                        def _():
                            scat(b - NBUF, par).wait()

                        conv(par)
                        scat(b, par).start()
                    return carry

                lax.fori_loop(0, NB_Q // NBUF, body, 0)
                for g in range(NB_Q - NBUF, NB_Q):
                    scat(g, g % NBUF).wait()
                return qcarry

            lax.fori_loop(0, NQ, qbody, 0)

            plsc.subcore_barrier()
            start = jnp.minimum(s * RPT, N - RPT)
            pltpu.sync_copy(
                acc.at[pl.ds(start, RPT)],
                out_hbm.at[pl.ds(off + start, RPT)],
            )
            plsc.subcore_barrier()

    return agg_kernel


_deg_call = functools.cache(_make_deg_kernel)
_agg_call = functools.cache(_make_agg_kernel)


def _tc_a_body(x_ref, d0_ref, d1_ref, hn_ref, hnp_ref, dinv_ref):
    deg = d0_ref[...] + d1_ref[...] + 1.0
    dinv = lax.rsqrt(deg)
    dinv_ref[...] = dinv
    for cch in range(IN_DIM // 128):
        hn = x_ref[:, cch * 128:(cch + 1) * 128] * dinv
        hn_ref[cch] = hn
        hnp_ref[cch] = _pack_bf16_rows(hn)


def _tc_a(x, d0, d1):
    return pl.pallas_call(
        _tc_a_body,
        grid=(NRB,),
        in_specs=[
            pl.BlockSpec((RB, IN_DIM), lambda i: (i, 0)),
            pl.BlockSpec((RB, 1), lambda i: (i, 0)),
            pl.BlockSpec((RB, 1), lambda i: (i, 0)),
        ],
        out_specs=[
            pl.BlockSpec((IN_DIM // 128, RB, 128), lambda i: (0, i, 0)),
            pl.BlockSpec((IN_DIM // 128, RB, 64), lambda i: (0, i, 0)),
            pl.BlockSpec((RB, 1), lambda i: (i, 0)),
        ],
        out_shape=[
            jax.ShapeDtypeStruct((IN_DIM // 128, N, 128), jnp.float32),
            jax.ShapeDtypeStruct((IN_DIM // 128, N, 64), jnp.int32),
            jax.ShapeDtypeStruct((N, 1), jnp.float32),
        ],
    )(x, d0, d1)


def _tc_layer_body(nk, nout_chunks, t_ref, hn_ref, dinv_ref, w_ref, b_ref,
                   out_ref, outp_ref):
    dinv = dinv_ref[...]
    acc = jnp.zeros((RB, HID), jnp.float32)
    for k in range(nk):
        a = (t_ref[k] + hn_ref[k]) * dinv
        acc = acc + jnp.dot(a, w_ref[k], preferred_element_type=jnp.float32)
    h = jnp.maximum(acc + b_ref[...], 0.0)
    hn = h * dinv
    for oc in range(nout_chunks):
        blk = hn[:, oc * 128:(oc + 1) * 128]
        out_ref[oc] = blk
        outp_ref[oc] = _pack_bf16_rows(blk)


def _tc_layer1(t1, hn1, dinv, w1r, b1):
    return pl.pallas_call(
        functools.partial(_tc_layer_body, 2, 4),
        grid=(NRB,),
        in_specs=[
            pl.BlockSpec((2, RB, 128), lambda i: (0, i, 0)),
            pl.BlockSpec((2, RB, 128), lambda i: (0, i, 0)),
            pl.BlockSpec((RB, 1), lambda i: (i, 0)),
            pl.BlockSpec((2, 128, HID), lambda i: (0, 0, 0)),
            pl.BlockSpec((1, HID), lambda i: (0, 0)),
        ],
        out_specs=[
            pl.BlockSpec((4, RB, 128), lambda i: (0, i, 0)),
            pl.BlockSpec((4, RB, 64), lambda i: (0, i, 0)),
        ],
        out_shape=[
            jax.ShapeDtypeStruct((4, N, 128), jnp.float32),
            jax.ShapeDtypeStruct((4, N, 64), jnp.int32),
        ],
    )(t1, hn1, dinv, w1r, b1)


def _tc_c_body(t_ref, hn_ref, dinv_ref, w_ref, b_ref, aw_ref, ab_ref,
               h2_ref, lg_ref):
    dinv = dinv_ref[...]
    acc = jnp.zeros((RB, HID), jnp.float32)
    for k in range(4):
        a = (t_ref[k] + hn_ref[k]) * dinv
        acc = acc + jnp.dot(a, w_ref[k], preferred_element_type=jnp.float32)
    h2 = jnp.maximum(acc + b_ref[...], 0.0)
    h2_ref[...] = h2
    lg_ref[...] = jnp.sum(h2 * aw_ref[...], axis=1, keepdims=True) + ab_ref[...]


def _tc_c(t2, hn2, dinv, w2r, b2, awr, ab):
    return pl.pallas_call(
        _tc_c_body,
        grid=(NRB,),
        in_specs=[
            pl.BlockSpec((4, RB, 128), lambda i: (0, i, 0)),
            pl.BlockSpec((4, RB, 128), lambda i: (0, i, 0)),
            pl.BlockSpec((RB, 1), lambda i: (i, 0)),
            pl.BlockSpec((4, 128, HID), lambda i: (0, 0, 0)),
            pl.BlockSpec((1, HID), lambda i: (0, 0)),
            pl.BlockSpec((1, HID), lambda i: (0, 0)),
            pl.BlockSpec((1, 1), lambda i: (0, 0)),
        ],
        out_specs=[
            pl.BlockSpec((RB, HID), lambda i: (i, 0)),
            pl.BlockSpec((RB, 1), lambda i: (i, 0)),
        ],
        out_shape=[
            jax.ShapeDtypeStruct((N, HID), jnp.float32),
            jax.ShapeDtypeStruct((N, 1), jnp.float32),
        ],
    )(t2, hn2, dinv, w2r, b2, awr, ab)


def _tc_d_body(h2_ref, lg_all_ref, lg_blk_ref, batch_ref, out_ref):
    i = pl.program_id(0)
    l_all = lg_all_ref[...]
    m = jnp.max(l_all)
    ssum = jnp.sum(jnp.exp(l_all - m))
    w = jnp.exp(lg_blk_ref[...] - m) / ssum
    z = h2_ref[...] * w
    bvec = batch_ref[0]
    gids = lax.broadcasted_iota(jnp.int32, (G, RB), 0)
    oh = (gids == bvec).astype(jnp.float32)
    contrib = jax.lax.dot_general(
        oh, z, (((1,), (0,)), ((), ())), preferred_element_type=jnp.float32
    )

    @pl.when(i == 0)
    def _():
        out_ref[...] = jnp.zeros_like(out_ref)

    out_ref[...] += contrib


def _tc_d(h2, lg, batch3):
    return pl.pallas_call(
        _tc_d_body,
        grid=(NRB,),
        in_specs=[
            pl.BlockSpec((RB, HID), lambda i: (i, 0)),
            pl.BlockSpec((N, 1), lambda i: (0, 0)),
            pl.BlockSpec((RB, 1), lambda i: (i, 0)),
            pl.BlockSpec((1, 1, RB), lambda i: (i, 0, 0)),
        ],
        out_specs=pl.BlockSpec((G, HID), lambda i: (0, 0)),
        out_shape=jax.ShapeDtypeStruct((G, HID), jnp.float32),
    )(h2, lg, lg, batch3)


def kernel(x, edge_index, batch, W1, b1, W2, b2, att_w, att_b):
    src = edge_index[0]
    dst = edge_index[1]
    pad = EP - E
    srcp = jnp.concatenate([src, jnp.zeros((pad,), jnp.int32)])
    dstp = jnp.concatenate([dst, jnp.full((pad,), N, jnp.int32)])
    src16 = srcp.reshape(NS * NQ, NB_Q, EB_A)
    dst16 = dstp.reshape(NS * NQ, NB_Q, EB_A)
    dst32 = dstp.reshape(NC * NS, NB_DEG, EB)

    ones128 = jnp.ones((EB, 128), jnp.float32)
    zeros128 = jnp.zeros((RPT, 128), jnp.float32)

    degp = _deg_call()(dst32, ones128, zeros128)
    d0 = degp[0:N, 0:1]
    d1 = degp[ACC_ROWS:ACC_ROWS + N, 0:1]

    hn1, hn1p, dinv = _tc_a(x, d0, d1)

    t1 = _agg_call(2)(hn1p.reshape(2 * N, 64), src16, dst16, zeros128)
    w1r = W1.reshape(2, 128, HID)
    hn2, hn2p = _tc_layer1(t1.reshape(2, N, 128), hn1, dinv, w1r,
                           b1.reshape(1, HID))

    t2 = _agg_call(4)(hn2p.reshape(4 * N, 64), src16, dst16, zeros128)
    w2r = W2.reshape(4, 128, HID)
    h2, lg = _tc_c(t2.reshape(4, N, 128), hn2, dinv, w2r, b2.reshape(1, HID),
                   att_w.reshape(1, HID), att_b.reshape(1, 1))

    pooled = _tc_d(h2, lg, batch.reshape(NRB, 1, RB))
    return pooled


# speedup vs baseline: 1.3223x; 1.3223x over previous
import functools

import jax
import jax.numpy as jnp
from jax import lax
from jax.experimental import pallas as pl
from jax.experimental.pallas import tpu as pltpu
from jax.experimental.pallas import tpu_sc as plsc

N = 10000
E = 160000
IN_DIM = 256
HID = 512
G = 128

NC, NS, LANES = 2, 16, 16
EB = 128
EP = 163840
EPT = EP // NS
NB_AGG = EPT // EB
EPW = EP // (NC * NS)
NB_DEG = EPW // EB
ACC_ROWS = 10112
RPT = ACC_ROWS // NS
RB = 1000
NRB = N // RB


def _sc_mesh():
    return plsc.VectorSubcoreMesh(
        core_axis_name="c", subcore_axis_name="s", num_cores=NC, num_subcores=NS
    )


def _make_deg_kernel():
    @functools.partial(
        pl.kernel,
        out_type=jax.ShapeDtypeStruct((NC * ACC_ROWS, 128), jnp.float32),
        mesh=_sc_mesh(),
        scratch_types=[
            pltpu.VMEM((NB_DEG, EB), jnp.int32),
            pltpu.VMEM((EB, 128), jnp.float32),
            pltpu.VMEM_SHARED((ACC_ROWS, 128), jnp.float32),
        ],
    )
    def deg_kernel(dst_hbm, ones_hbm, zeros_hbm, out_hbm, dst_v, ones_v, acc):
        c = lax.axis_index("c")
        s = lax.axis_index("s")
        w = c * NS + s
        pltpu.sync_copy(dst_hbm.at[w], dst_v)
        pltpu.sync_copy(ones_hbm, ones_v)
        pltpu.sync_copy(zeros_hbm, acc.at[pl.ds(s * RPT, RPT)])
        plsc.subcore_barrier()

        def body(b, carry):
            pltpu.sync_copy(ones_v, acc.at[dst_v.at[b]], add=True)
            return carry

        lax.fori_loop(0, NB_DEG, body, 0)
        plsc.subcore_barrier()
        pltpu.sync_copy(
            acc.at[pl.ds(s * RPT, RPT)],
            out_hbm.at[pl.ds(c * ACC_ROWS + s * RPT, RPT)],
        )

    return deg_kernel


def _make_agg_kernel(n_chunks):
    cpc = n_chunks // NC

    @functools.partial(
        pl.kernel,
        out_type=jax.ShapeDtypeStruct((n_chunks * N, 128), jnp.float32),
        mesh=_sc_mesh(),
        scratch_types=[
            pltpu.VMEM((NB_DEG, EB), jnp.int32),
            pltpu.VMEM((NB_DEG, EB), jnp.int32),
            pltpu.VMEM((EB, 128), jnp.float32),
            pltpu.VMEM((EB, 128), jnp.float32),
            pltpu.VMEM_SHARED((ACC_ROWS, 128), jnp.float32),
            pltpu.SemaphoreType.DMA,
            pltpu.SemaphoreType.DMA,
        ],
    )
    def agg_kernel(table_hbm, src_hbm, dst_hbm, zeros_hbm, out_hbm,
                   src_v, dst_v, buf_a, buf_b, acc, sem_a, sem_b):
        c = lax.axis_index("c")
        s = lax.axis_index("s")

        for j in range(cpc):
            off = (c * cpc + j) * N
            pltpu.sync_copy(zeros_hbm, acc.at[pl.ds(s * RPT, RPT)])
            plsc.subcore_barrier()

            for h in range(2):
                row = 2 * s + h
                pltpu.sync_copy(src_hbm.at[row], src_v)
                pltpu.sync_copy(dst_hbm.at[row], dst_v)

                def obody(i, carry):
                    r = i // (EB // LANES)
                    col = (i % (EB // LANES)) * LANES
                    src_v[r, pl.ds(col, LANES)] = (
                        src_v[r, pl.ds(col, LANES)] + off
                    )
                    return carry

                lax.fori_loop(0, NB_DEG * (EB // LANES), obody, 0)

                pltpu.async_copy(table_hbm.at[src_v.at[0]], buf_a, sem_a)

                def body(bb, carry):
                    for par, bufp, semp, bufn, semn in (
                        (0, buf_a, sem_a, buf_b, sem_b),
                        (1, buf_b, sem_b, buf_a, sem_a),
                    ):
                        b = bb * 2 + par

                        @pl.when(b + 1 < NB_DEG)
                        def _():
                            pltpu.async_copy(
                                table_hbm.at[src_v.at[b + 1]], bufn, semn
                            )

                        pltpu.make_async_copy(
                            table_hbm.at[src_v.at[b]], bufp, semp
                        ).wait()
                        pltpu.sync_copy(bufp, acc.at[dst_v.at[b]], add=True)
                    return carry

                lax.fori_loop(0, NB_DEG // 2, body, 0)

            plsc.subcore_barrier()
            start = jnp.minimum(s * RPT, N - RPT)
            pltpu.sync_copy(
                acc.at[pl.ds(start, RPT)],
                out_hbm.at[pl.ds(off + start, RPT)],
            )
            plsc.subcore_barrier()

    return agg_kernel


_deg_call = functools.cache(_make_deg_kernel)
_agg_call = functools.cache(_make_agg_kernel)


def _tc_a_body(x_ref, d0_ref, d1_ref, hn_ref, dinv_ref):
    deg = d0_ref[...] + d1_ref[...] + 1.0
    dinv = lax.rsqrt(deg)
    dinv_ref[...] = dinv
    for cch in range(IN_DIM // 128):
        hn_ref[cch] = x_ref[:, cch * 128:(cch + 1) * 128] * dinv


def _tc_a(x, d0, d1):
    return pl.pallas_call(
        _tc_a_body,
        grid=(NRB,),
        in_specs=[
            pl.BlockSpec((RB, IN_DIM), lambda i: (i, 0)),
            pl.BlockSpec((RB, 1), lambda i: (i, 0)),
            pl.BlockSpec((RB, 1), lambda i: (i, 0)),
        ],
        out_specs=[
            pl.BlockSpec((IN_DIM // 128, RB, 128), lambda i: (0, i, 0)),
            pl.BlockSpec((RB, 1), lambda i: (i, 0)),
        ],
        out_shape=[
            jax.ShapeDtypeStruct((IN_DIM // 128, N, 128), jnp.float32),
            jax.ShapeDtypeStruct((N, 1), jnp.float32),
        ],
    )(x, d0, d1)


def _tc_layer_body(nk, nout_chunks, t_ref, hn_ref, dinv_ref, w_ref, b_ref,
                   out_ref, h_ref=None):
    dinv = dinv_ref[...]
    acc = jnp.zeros((RB, HID), jnp.float32)
    for k in range(nk):
        a = (t_ref[k] + hn_ref[k]) * dinv
        acc = acc + jnp.dot(a, w_ref[k], preferred_element_type=jnp.float32)
    h = jnp.maximum(acc + b_ref[...], 0.0)
    if h_ref is not None:
        h_ref[...] = h
    hn = h * dinv
    for oc in range(nout_chunks):
        out_ref[oc] = hn[:, oc * 128:(oc + 1) * 128]


def _tc_layer1(t1, hn1, dinv, w1r, b1):
    return pl.pallas_call(
        functools.partial(_tc_layer_body, 2, 4),
        grid=(NRB,),
        in_specs=[
            pl.BlockSpec((2, RB, 128), lambda i: (0, i, 0)),
            pl.BlockSpec((2, RB, 128), lambda i: (0, i, 0)),
            pl.BlockSpec((RB, 1), lambda i: (i, 0)),
            pl.BlockSpec((2, 128, HID), lambda i: (0, 0, 0)),
            pl.BlockSpec((1, HID), lambda i: (0, 0)),
        ],
        out_specs=pl.BlockSpec((4, RB, 128), lambda i: (0, i, 0)),
        out_shape=jax.ShapeDtypeStruct((4, N, 128), jnp.float32),
    )(t1, hn1, dinv, w1r, b1)


def _tc_cd_body(t_ref, hn_ref, dinv_ref, w_ref, b_ref, aw_ref, ab_ref,
                batch_ref, out_ref, h2_scr, lg_scr):
    i = pl.program_id(0)

    @pl.when(i < NRB)
    def _():
        dinv = dinv_ref[...]
        acc = jnp.zeros((RB, HID), jnp.float32)
        for k in range(4):
            a = (t_ref[k] + hn_ref[k]) * dinv
            acc = acc + jnp.dot(a, w_ref[k],
                                preferred_element_type=jnp.float32)
        h2 = jnp.maximum(acc + b_ref[...], 0.0)
        h2_scr[pl.ds(i * RB, RB), :] = h2
        lg_scr[pl.ds(i * RB, RB), :] = (
            jnp.sum(h2 * aw_ref[...], axis=1, keepdims=True) + ab_ref[...]
        )

    @pl.when(i >= NRB)
    def _():
        ii = i - NRB
        l_all = lg_scr[...]
        m = jnp.max(l_all)
        ssum = jnp.sum(jnp.exp(l_all - m))
        lblk = lg_scr[pl.ds(ii * RB, RB), :]
        w = jnp.exp(lblk - m) / ssum
        z = h2_scr[pl.ds(ii * RB, RB), :] * w
        bvec = batch_ref[0]
        gids = lax.broadcasted_iota(jnp.int32, (G, RB), 0)
        oh = (gids == bvec).astype(jnp.float32)
        contrib = jax.lax.dot_general(
            oh, z, (((1,), (0,)), ((), ())),
            preferred_element_type=jnp.float32)

        @pl.when(ii == 0)
        def _():
            out_ref[...] = jnp.zeros_like(out_ref)

        out_ref[...] += contrib


def _tc_cd(t2, hn2, dinv, w2r, b2, awr, ab, batch3):
    clamp = lambda i: jnp.minimum(i, NRB - 1)
    return pl.pallas_call(
        _tc_cd_body,
        grid=(2 * NRB,),
        in_specs=[
            pl.BlockSpec((4, RB, 128), lambda i: (0, clamp(i), 0)),
            pl.BlockSpec((4, RB, 128), lambda i: (0, clamp(i), 0)),
            pl.BlockSpec((RB, 1), lambda i: (clamp(i), 0)),
            pl.BlockSpec((4, 128, HID), lambda i: (0, 0, 0)),
            pl.BlockSpec((1, HID), lambda i: (0, 0)),
            pl.BlockSpec((1, HID), lambda i: (0, 0)),
            pl.BlockSpec((1, 1), lambda i: (0, 0)),
            pl.BlockSpec((1, 1, RB), lambda i: (jnp.maximum(i - NRB, 0), 0, 0)),
        ],
        out_specs=pl.BlockSpec((G, HID), lambda i: (0, 0)),
        out_shape=jax.ShapeDtypeStruct((G, HID), jnp.float32),
        scratch_shapes=[
            pltpu.VMEM((N, HID), jnp.float32),
            pltpu.VMEM((N, 1), jnp.float32),
        ],
    )(t2, hn2, dinv, w2r, b2, awr, ab, batch3)


def kernel(x, edge_index, batch, W1, b1, W2, b2, att_w, att_b):
    src = edge_index[0]
    dst = edge_index[1]
    pad = EP - E
    srcp = jnp.concatenate([src, jnp.zeros((pad,), jnp.int32)])
    dstp = jnp.concatenate([dst, jnp.full((pad,), N, jnp.int32)])
    src32 = srcp.reshape(NC * NS, NB_DEG, EB)
    dst32 = dstp.reshape(NC * NS, NB_DEG, EB)

    ones128 = jnp.ones((EB, 128), jnp.float32)
    zeros128 = jnp.zeros((RPT, 128), jnp.float32)

    degp = _deg_call()(dst32, ones128, zeros128)
    d0 = degp[0:N, 0:1]
    d1 = degp[ACC_ROWS:ACC_ROWS + N, 0:1]

    hn1, dinv = _tc_a(x, d0, d1)

    t1 = _agg_call(2)(hn1.reshape(2 * N, 128), src32, dst32, zeros128)
    w1r = W1.reshape(2, 128, HID)
    hn2 = _tc_layer1(t1.reshape(2, N, 128), hn1, dinv, w1r, b1.reshape(1, HID))

    t2 = _agg_call(4)(hn2.reshape(4 * N, 128), src32, dst32, zeros128)
    w2r = W2.reshape(4, 128, HID)
    pooled = _tc_cd(t2.reshape(4, N, 128), hn2, dinv, w2r,
                    b2.reshape(1, HID), att_w.reshape(1, HID),
                    att_b.reshape(1, 1), batch.reshape(NRB, 1, RB))
    return pooled
